# Initial kernel scaffold; baseline (speedup 1.0000x reference)
import jax
import jax.numpy as jnp
from jax.experimental import pallas as pl


def kernel(x, edge_index_onset, edge_index_consecutive, edge_index_sustain, edge_index_silence, W1_onset, b1_onset, W2_onset, b2_onset, W1_consecutive, b1_consecutive, W2_consecutive, b2_consecutive, W1_sustain, b1_sustain, W2_sustain, b2_sustain, W1_silence, b1_silence, W2_silence, b2_silence):
    raise NotImplementedError("write your pallas kernel here")


# speedup vs baseline: 10.9918x; 10.9918x over previous
import jax
import jax.numpy as jnp
from jax import lax
from jax.experimental import pallas as pl
from jax.experimental.pallas import tpu as pltpu
from jax.experimental.pallas import tpu_sc as plsc

N = 100000
E = 1600000
NC = 2
NS = 16
SUB = 128
M = 8
ROWS_T = 800
ROWS_R = NS * ROWS_T
EP = ROWS_R * SUB
NCHUNK = ROWS_T // M
NT = 100096
STRIPE = NT // NS
BLK = STRIPE
NBLK = NT // BLK


def _agg_body(xp, srcs, dsts, agg, acc, sidx, didx, rbuf, gsem, ssem):
    c = lax.axis_index("c")
    t = lax.axis_index("s")
    row0 = t * STRIPE

    def init_acc():
        pltpu.sync_copy(xp.at[pl.ds(c * NT + row0, STRIPE)],
                        acc.at[pl.ds(row0, STRIPE)])

    init_acc()
    plsc.subcore_barrier()
    for r in range(4):
        sbase = c * (4 * ROWS_R) + r * ROWS_R + t * ROWS_T
        dbase = r * ROWS_R + t * ROWS_T

        def chunk(i, _, sbase=sbase, dbase=dbase):
            pltpu.sync_copy(srcs.at[pl.ds(sbase + i * M, M)], sidx)
            pltpu.sync_copy(dsts.at[pl.ds(dbase + i * M, M)], didx)

            def fire_g(j, _):
                pltpu.async_copy(xp.at[sidx.at[j]], rbuf.at[j], gsem)
                return 0

            def wait_g(j, _):
                pltpu.make_async_copy(xp.at[sidx.at[j]], rbuf.at[j],
                                      gsem).wait()
                return 0

            def fire_s(j, _):
                pltpu.async_copy(rbuf.at[j], acc.at[didx.at[j]], ssem,
                                 add=True)
                return 0

            def wait_s(j, _):
                pltpu.make_async_copy(rbuf.at[j], acc.at[didx.at[j]],
                                      ssem).wait()
                return 0

            lax.fori_loop(0, M, fire_g, 0)
            lax.fori_loop(0, M, wait_g, 0)
            lax.fori_loop(0, M, fire_s, 0)
            lax.fori_loop(0, M, wait_s, 0)
            return 0

        lax.fori_loop(0, NCHUNK, chunk, 0)
        plsc.subcore_barrier()
        pltpu.sync_copy(acc.at[pl.ds(row0, STRIPE)],
                        agg.at[pl.ds((c * 4 + r) * NT + row0, STRIPE)])
        if r < 3:
            init_acc()
        plsc.subcore_barrier()


_agg_call = pl.kernel(
    _agg_body,
    out_type=jax.ShapeDtypeStruct((8 * NT, 16), jnp.float32),
    compiler_params=pltpu.CompilerParams(use_tc_tiling_on_sc=False),
    mesh=plsc.VectorSubcoreMesh(core_axis_name="c", subcore_axis_name="s"),
    scratch_types=[
        pltpu.VMEM_SHARED((NT, 16), jnp.float32),
        pltpu.VMEM((M, SUB), jnp.int32),
        pltpu.VMEM((M, SUB), jnp.int32),
        pltpu.VMEM((M, SUB, 16), jnp.float32),
        pltpu.SemaphoreType.DMA,
        pltpu.SemaphoreType.DMA,
    ],
)


def _vw_body(cflat, dsts, src3, vout, w, ctile, sidx, didx, vbuf, ssem):
    c = lax.axis_index("c")
    t = lax.axis_index("s")
    s0 = t * STRIPE

    for rr in range(2):
        r = c * 2 + rr
        pltpu.sync_copy(cflat.at[pl.ds(r * NT + s0, STRIPE)],
                        w.at[pl.ds(rr * NT + s0, STRIPE)])
    plsc.subcore_barrier()
    for rr in range(2):
        r = c * 2 + rr
        pltpu.sync_copy(cflat.at[pl.ds(r * NT, NT)], ctile)
        base = r * ROWS_R + t * ROWS_T

        def chunk(i, _, base=base):
            pltpu.sync_copy(dsts.at[pl.ds(base + i * M, M)], didx)
            pltpu.sync_copy(src3.at[pl.ds(base + i * M, M)], sidx)

            def gath(j, _):
                def lanes(k, _):
                    idxv = didx[j, pl.ds(k * 16, 16)]
                    vbuf[j, pl.ds(k * 16, 16)] = plsc.load_gather(
                        ctile, [idxv])
                    return 0
                lax.fori_loop(0, SUB // 16, lanes, 0)
                return 0

            def fire(j, _):
                pltpu.async_copy(vbuf.at[j], w.at[sidx.at[j]], ssem,
                                 add=True)
                return 0

            def drain(j, _):
                pltpu.make_async_copy(vbuf.at[j], w.at[sidx.at[j]],
                                      ssem).wait()
                return 0

            lax.fori_loop(0, M, gath, 0)
            lax.fori_loop(0, M, fire, 0)
            lax.fori_loop(0, M, drain, 0)
            return 0

        lax.fori_loop(0, NCHUNK, chunk, 0)
    plsc.subcore_barrier()
    for rr in range(2):
        r = c * 2 + rr
        pltpu.sync_copy(w.at[pl.ds(rr * NT + s0, STRIPE)],
                        vout.at[pl.ds(r * NT + s0, STRIPE)])


_vw_call = pl.kernel(
    _vw_body,
    out_type=jax.ShapeDtypeStruct((4 * NT,), jnp.float32),
    compiler_params=pltpu.CompilerParams(use_tc_tiling_on_sc=False,
                                         needs_layout_passes=False),
    mesh=plsc.VectorSubcoreMesh(core_axis_name="c", subcore_axis_name="s"),
    scratch_types=[
        pltpu.VMEM_SHARED((2 * NT,), jnp.float32),
        pltpu.VMEM((NT,), jnp.float32),
        pltpu.VMEM((M, SUB), jnp.int32),
        pltpu.VMEM((M, SUB), jnp.int32),
        pltpu.VMEM((M, SUB), jnp.float32),
        pltpu.SemaphoreType.DMA,
    ],
)


def _recip_body(d_ref, o_ref):
    o_ref[...] = 1.0 / jnp.maximum(d_ref[...], 1e-30)


def _recip(degp1):
    return pl.pallas_call(
        _recip_body,
        grid=(NBLK,),
        in_specs=[pl.BlockSpec((4, 1, 1, BLK), lambda i: (0, i, 0, 0))],
        out_specs=pl.BlockSpec((4, 1, 1, BLK), lambda i: (0, i, 0, 0)),
        out_shape=jax.ShapeDtypeStruct((4, NBLK, 1, BLK), jnp.float32),
    )(degp1.reshape(4, NBLK, 1, BLK))


def _final_body(agg_ref, c_ref, v_ref, w1_ref, b1_ref, w2_ref, b2_ref,
                o_ref, s_ref):
    i = pl.program_id(0)

    @pl.when(i == 0)
    def _():
        s_ref[...] = jnp.zeros_like(s_ref)

    cc = c_ref[...]
    t = jnp.zeros((BLK, 32), jnp.float32)
    for r in range(4):
        ar = (jnp.dot(agg_ref[0, r], w1_ref[r, 0],
                      preferred_element_type=jnp.float32)
              + jnp.dot(agg_ref[1, r], w1_ref[r, 1],
                        preferred_element_type=jnp.float32))
        t = t + ar * cc[r, 0, 0][:, None]
    h = jnp.maximum(t + b1_ref[...], 0.0)
    rowid = lax.broadcasted_iota(jnp.int32, (BLK, 1), 0) + i * BLK
    h = jnp.where(rowid < N, h, 0.0)
    vv = v_ref[...].reshape(4, BLK)
    s_ref[...] += jnp.dot(vv, h, preferred_element_type=jnp.float32)

    @pl.when(i == NBLK - 1)
    def _():
        s = s_ref[...]
        o = jnp.zeros((1, 64), jnp.float32)
        for r in range(4):
            o = o + jnp.dot(s[r][None, :], w2_ref[r],
                            preferred_element_type=jnp.float32)
        o_ref[...] = o * (1.0 / N) + b2_ref[...]


def _final(agg, c4, v4, w1p, b1s, w2s, b2s):
    return pl.pallas_call(
        _final_body,
        grid=(NBLK,),
        in_specs=[
            pl.BlockSpec((2, 4, BLK, 16), lambda i: (0, 0, i, 0)),
            pl.BlockSpec((4, 1, 1, BLK), lambda i: (0, i, 0, 0)),
            pl.BlockSpec((4, 1, 1, BLK), lambda i: (0, i, 0, 0)),
            pl.BlockSpec((4, 2, 16, 32), lambda i: (0, 0, 0, 0)),
            pl.BlockSpec((1, 32), lambda i: (0, 0)),
            pl.BlockSpec((4, 32, 64), lambda i: (0, 0, 0)),
            pl.BlockSpec((1, 64), lambda i: (0, 0)),
        ],
        out_specs=pl.BlockSpec((1, 64), lambda i: (0, 0)),
        out_shape=jax.ShapeDtypeStruct((1, 64), jnp.float32),
        scratch_shapes=[pltpu.VMEM((4, 32), jnp.float32)],
        compiler_params=pltpu.CompilerParams(
            vmem_limit_bytes=100 * 1024 * 1024),
    )(agg, c4, v4, w1p, b1s, w2s, b2s)


def _pad_idx(idx):
    npad = EP - E
    pad = (jnp.arange(npad, dtype=jnp.int32) % 8) + N
    return jnp.concatenate([idx, pad])


def kernel(x, edge_index_onset, edge_index_consecutive, edge_index_sustain,
           edge_index_silence, W1_onset, b1_onset, W2_onset, b2_onset,
           W1_consecutive, b1_consecutive, W2_consecutive, b2_consecutive,
           W1_sustain, b1_sustain, W2_sustain, b2_sustain, W1_silence,
           b1_silence, W2_silence, b2_silence):
    edges = [edge_index_onset, edge_index_consecutive, edge_index_sustain,
             edge_index_silence]
    W1s = [W1_onset, W1_consecutive, W1_sustain, W1_silence]
    b1s = [b1_onset, b1_consecutive, b1_sustain, b1_silence]
    W2s = [W2_onset, W2_consecutive, W2_sustain, W2_silence]
    b2s = [b2_onset, b2_consecutive, b2_sustain, b2_silence]

    f32 = jnp.float32
    zcol = jnp.zeros((N, 4), f32)
    zpad = jnp.zeros((NT - N, 16), f32)
    h0 = jnp.concatenate([x[:, :12], zcol], axis=1)
    h1 = jnp.concatenate([x[:, 12:23], jnp.ones((N, 1), f32), zcol], axis=1)
    xp = jnp.concatenate([h0, zpad, h1, zpad], axis=0)

    src_p = [_pad_idx(e[0]) for e in edges]
    dst_p = [_pad_idx(e[1]) for e in edges]
    src_cat = jnp.concatenate(src_p)
    srcs = jnp.concatenate([src_cat, src_cat + NT]).reshape(8 * ROWS_R, SUB)
    dsts = jnp.concatenate(dst_p).reshape(4 * ROWS_R, SUB)
    src3 = jnp.concatenate(
        [src_p[r] + (r % 2) * NT for r in range(4)]).reshape(4 * ROWS_R, SUB)

    agg = _agg_call(xp, srcs, dsts)
    aggr = agg.reshape(2, 4, NT, 16)
    degp1 = aggr[1, :, :, 11]
    c4 = _recip(degp1)
    cflat = c4.reshape(-1)
    vout = _vw_call(cflat, dsts, src3)
    v4 = vout.reshape(4, NBLK, 1, BLK)

    w1p = jnp.stack([
        jnp.stack([
            jnp.concatenate([W1s[r][:12], jnp.zeros((4, 32), f32)], axis=0),
            jnp.concatenate([W1s[r][12:23], jnp.zeros((5, 32), f32)], axis=0),
        ])
        for r in range(4)
    ])
    b1sum = (b1s[0] + b1s[1] + b1s[2] + b1s[3]).reshape(1, 32)
    w2s = jnp.stack(W2s)
    b2sum = (b2s[0] + b2s[1] + b2s[2] + b2s[3]).reshape(1, 64)

    return _final(aggr, c4, v4, w1p, b1sum, w2s, b2sum)
